# baseline (speedup 1.0000x reference)
import functools

import jax
import jax.numpy as jnp
from jax import lax
from jax.experimental import pallas as pl
from jax.experimental.pallas import tpu as pltpu
from jax.experimental.pallas import tpu_sc as plsc

EMBED_DIM = 32
B_TOTAL = 16384 * 26
NC, NS = 2, 16
NW = NC * NS
B_PER_W = B_TOTAL // NW
CHUNK = 832
N_CHUNKS = B_PER_W // CHUNK
NBUF = 4
LOOKAHEAD = 2

_mesh = plsc.VectorSubcoreMesh(core_axis_name="c", subcore_axis_name="s")


@functools.partial(
    pl.kernel,
    mesh=_mesh,
    out_type=jax.ShapeDtypeStruct((B_TOTAL, EMBED_DIM), jnp.float32),
    scratch_types=[
        pltpu.VMEM((B_PER_W,), jnp.int32),
        pltpu.VMEM((NBUF, CHUNK, EMBED_DIM), jnp.float32),
        [pltpu.SemaphoreType.DMA] * NBUF,
        [pltpu.SemaphoreType.DMA] * NBUF,
    ],
    compiler_params=pltpu.CompilerParams(use_tc_tiling_on_sc=False),
)
def _embedding_gather(idx_hbm, table_hbm, out_hbm, idx_v, rows_v, sems_g, sems_w):
    wid = lax.axis_index("s") * NC + lax.axis_index("c")
    base = wid * B_PER_W
    pltpu.sync_copy(idx_hbm.at[pl.ds(base, B_PER_W)], idx_v)

    gathers = [None] * N_CHUNKS
    writes = [None] * N_CHUNKS

    def fire_gather(c):
        slot = c % NBUF
        gathers[c] = pltpu.async_copy(
            table_hbm.at[idx_v.at[pl.ds(c * CHUNK, CHUNK)]],
            rows_v.at[slot],
            sems_g[slot],
        )

    for c in range(min(LOOKAHEAD, N_CHUNKS)):
        fire_gather(c)
    for c in range(N_CHUNKS):
        nc = c + LOOKAHEAD
        if nc < N_CHUNKS:
            if nc >= NBUF:
                writes[nc - NBUF].wait()
            fire_gather(nc)
        gathers[c].wait()
        slot = c % NBUF
        writes[c] = pltpu.async_copy(
            rows_v.at[slot],
            out_hbm.at[pl.ds(base + c * CHUNK, CHUNK)],
            sems_w[slot],
        )
    for c in range(max(0, N_CHUNKS - NBUF), N_CHUNKS):
        writes[c].wait()


B = 16384
F = 26
B_PER_W2 = B // NW


@functools.partial(
    pl.kernel,
    mesh=_mesh,
    out_type=jax.ShapeDtypeStruct((B_TOTAL,), jnp.int32),
    scratch_types=[pltpu.VMEM((B_PER_W2,), jnp.int32)],
    compiler_params=pltpu.CompilerParams(use_tc_tiling_on_sc=True),
)
def _idx_detile(idxT_hbm, out_hbm, buf_v):
    wid = lax.axis_index("s") * NC + lax.axis_index("c")
    b0 = wid * B_PER_W2
    for f in range(F):
        pltpu.sync_copy(idxT_hbm.at[f, pl.ds(b0, B_PER_W2)], buf_v)
        pltpu.sync_copy(buf_v, out_hbm.at[pl.ds(f * B + b0, B_PER_W2)])


def kernel(input, weight):
    idx_flat = _idx_detile(input.T)
    out = _embedding_gather(idx_flat, weight)
    out3 = out.reshape(input.shape[1], input.shape[0], EMBED_DIM)
    return jnp.transpose(out3, (1, 0, 2))


# speedup vs baseline: 1.0049x; 1.0049x over previous
import functools

import jax
import jax.numpy as jnp
from jax import lax
from jax.experimental import pallas as pl
from jax.experimental.pallas import tpu as pltpu
from jax.experimental.pallas import tpu_sc as plsc

EMBED_DIM = 32
B = 16384
F = 26
NC, NS = 2, 16
NW = NC * NS
CHUNK = B // NW
NBUF = 4
LOOKAHEAD = 2

_mesh = plsc.VectorSubcoreMesh(core_axis_name="c", subcore_axis_name="s")


@functools.partial(
    pl.kernel,
    mesh=_mesh,
    out_type=jax.ShapeDtypeStruct((F, B, EMBED_DIM), jnp.float32),
    scratch_types=[
        pltpu.VMEM((NBUF, CHUNK), jnp.int32),
        pltpu.VMEM((NBUF, CHUNK, EMBED_DIM), jnp.float32),
        [pltpu.SemaphoreType.DMA] * NBUF,
        [pltpu.SemaphoreType.DMA] * NBUF,
        [pltpu.SemaphoreType.DMA] * NBUF,
    ],
    compiler_params=pltpu.CompilerParams(use_tc_tiling_on_sc=False),
)
def _embedding_gather(idxT_hbm, table_hbm, out_hbm, idx_v, rows_v, sems_i,
                      sems_g, sems_w):
    wid = lax.axis_index("s") * NC + lax.axis_index("c")
    b0 = wid * CHUNK

    idx_loads = [None] * F
    gathers = [None] * F
    writes = [None] * F

    def fire_idx(f):
        slot = f % NBUF
        idx_loads[f] = pltpu.async_copy(
            idxT_hbm.at[f, pl.ds(b0, CHUNK)], idx_v.at[slot], sems_i[slot]
        )

    def fire_gather(f):
        slot = f % NBUF
        idx_loads[f].wait()
        gathers[f] = pltpu.async_copy(
            table_hbm.at[idx_v.at[slot]], rows_v.at[slot], sems_g[slot]
        )

    for f in range(min(LOOKAHEAD, F)):
        fire_idx(f)
    for f in range(min(LOOKAHEAD, F)):
        fire_gather(f)
    for f in range(F):
        nf = f + LOOKAHEAD
        if nf < F:
            if nf >= NBUF:
                writes[nf - NBUF].wait()
            fire_idx(nf)
            fire_gather(nf)
        gathers[f].wait()
        slot = f % NBUF
        writes[f] = pltpu.async_copy(
            rows_v.at[slot], out_hbm.at[f, pl.ds(b0, CHUNK)], sems_w[slot]
        )
    for f in range(max(0, F - NBUF), F):
        writes[f].wait()


def kernel(input, weight):
    out3 = _embedding_gather(input.T, weight)
    return jnp.transpose(out3, (1, 0, 2))
